# initial kernel scaffold (unmeasured)
import jax
import jax.numpy as jnp
from jax import lax
from jax.experimental import pallas as pl
from jax.experimental.pallas import tpu as pltpu

N_DEV = 4
SQ = 256
SKV = 4096
HQ_PER = 8
DH = 128
DM = 1024
BLK = 64
SCALE = 0.08838834764831843


def kernel(x, Wq, K_ext, V_ext, Wo):
    x_my = x[0].astype(jnp.bfloat16)
    wq = Wq.astype(jnp.bfloat16)
    wo = Wo.astype(jnp.bfloat16)

    def body(x_ref, wq_ref, wo_ref, k_ref, v_ref, out_ref,
             xbuf, accbuf, kbuf, vbuf,
             xsend, xrecv, asend, arecv, ksem, vsem):
        my = lax.axis_index("i")
        left = lax.rem(my + N_DEV - 1, N_DEV)
        right = lax.rem(my + 1, N_DEV)
        h0 = my * HQ_PER

        barrier = pltpu.get_barrier_semaphore()
        for nbr in (left, right):
            pl.semaphore_signal(barrier, inc=1, device_id=(nbr,),
                                device_id_type=pl.DeviceIdType.MESH)
        pl.semaphore_wait(barrier, 2)

        qb = lax.broadcasted_iota(jnp.int32, (SQ, SKV), 0) // BLK
        kb = lax.broadcasted_iota(jnp.int32, (SQ, SKV), 1) // BLK
        mask = (qb == kb) | (kb == 0) | (lax.rem(qb + kb, 3) == 0)
        bias = jnp.where(mask, 0.0, -1e9).astype(jnp.float32)

        def compute_partial(xv, b):
            ck = pltpu.make_async_copy(
                k_ref.at[b, :, pl.ds(h0, HQ_PER), :], kbuf, ksem)
            cv = pltpu.make_async_copy(
                v_ref.at[b, :, pl.ds(h0, HQ_PER), :], vbuf, vsem)
            ck.start()
            cv.start()
            q = jnp.dot(xv, wq_ref[:, :],
                        preferred_element_type=jnp.float32)
            q = q.astype(jnp.bfloat16)
            ck.wait()
            cv.wait()
            ctxs = []
            for hh in range(HQ_PER):
                qh = q[:, hh * DH:(hh + 1) * DH]
                kh = kbuf[:, hh, :].astype(jnp.bfloat16)
                s = lax.dot_general(qh, kh, (((1,), (1,)), ((), ())),
                                    preferred_element_type=jnp.float32)
                s = s * SCALE + bias
                m = jnp.max(s, axis=1, keepdims=True)
                w = jnp.exp(s - m)
                den = jnp.sum(w, axis=1, keepdims=True)
                w = (w / den).astype(jnp.bfloat16)
                vh = vbuf[:, hh, :].astype(jnp.bfloat16)
                ctxs.append(jnp.dot(w, vh,
                                    preferred_element_type=jnp.float32))
            ctx = jnp.concatenate(ctxs, axis=1).astype(jnp.bfloat16)
            return jnp.dot(ctx, wo_ref[:, :],
                           preferred_element_type=jnp.float32)

        xbuf[0] = x_ref[:, :]
        accbuf[0] = compute_partial(x_ref[:, :], my)

        for h in range(N_DEV - 1):
            rx = pltpu.make_async_remote_copy(
                src_ref=xbuf.at[h], dst_ref=xbuf.at[h + 1],
                send_sem=xsend.at[h], recv_sem=xrecv.at[h + 1],
                device_id=(right,), device_id_type=pl.DeviceIdType.MESH)
            ra = pltpu.make_async_remote_copy(
                src_ref=accbuf.at[h], dst_ref=accbuf.at[h + 1],
                send_sem=asend.at[h], recv_sem=arecv.at[h + 1],
                device_id=(right,), device_id_type=pl.DeviceIdType.MESH)
            rx.start()
            ra.start()
            rx.wait()
            ra.wait()
            b = lax.rem(my - h - 1 + N_DEV, N_DEV)
            p = compute_partial(xbuf[h + 1], b)
            accbuf[h + 1] = accbuf[h + 1] + p

        rf = pltpu.make_async_remote_copy(
            src_ref=accbuf.at[N_DEV - 1], dst_ref=accbuf.at[0],
            send_sem=asend.at[N_DEV - 1], recv_sem=arecv.at[0],
            device_id=(right,), device_id_type=pl.DeviceIdType.MESH)
        rf.start()
        rf.wait()
        out_ref[0] = accbuf[0]

    out_shape = jax.ShapeDtypeStruct((1, SQ, DM), jnp.float32)
    return pl.pallas_call(
        body,
        out_shape=out_shape,
        in_specs=[
            pl.BlockSpec(memory_space=pltpu.VMEM),
            pl.BlockSpec(memory_space=pltpu.VMEM),
            pl.BlockSpec(memory_space=pltpu.VMEM),
            pl.BlockSpec(memory_space=pltpu.ANY),
            pl.BlockSpec(memory_space=pltpu.ANY),
        ],
        out_specs=pl.BlockSpec(memory_space=pltpu.VMEM),
        scratch_shapes=[
            pltpu.VMEM((N_DEV, SQ, DM), jnp.bfloat16),
            pltpu.VMEM((N_DEV, SQ, DM), jnp.float32),
            pltpu.VMEM((SKV, HQ_PER, DH), jnp.float32),
            pltpu.VMEM((SKV, HQ_PER, DH), jnp.float32),
            pltpu.SemaphoreType.DMA((N_DEV,)),
            pltpu.SemaphoreType.DMA((N_DEV,)),
            pltpu.SemaphoreType.DMA((N_DEV,)),
            pltpu.SemaphoreType.DMA((N_DEV,)),
            pltpu.SemaphoreType.DMA,
            pltpu.SemaphoreType.DMA,
        ],
        compiler_params=pltpu.CompilerParams(collective_id=0),
    )(x_my, wq, wo, K_ext, V_ext)


# baseline (device time: 217742 ns/iter reference)
import jax
import jax.numpy as jnp
from jax import lax
from jax.experimental import pallas as pl
from jax.experimental.pallas import tpu as pltpu

N_DEV = 4
SQ = 256
SKV = 4096
HQ_PER = 8
DH = 128
DM = 1024
BLK = 64
SCALE = 0.08838834764831843


def kernel(x, Wq, K_ext, V_ext, Wo):
    x_my = x[0].astype(jnp.bfloat16)
    wq3 = Wq.astype(jnp.bfloat16).reshape(1024, HQ_PER, DH)
    wq3 = wq3.transpose(1, 0, 2)
    wo3 = Wo.astype(jnp.bfloat16).reshape(HQ_PER, DH, DM)

    def body(x_ref, wq_ref, wo_ref, k_ref, v_ref, out_ref,
             xbuf, accbuf, kbuf, vbuf,
             xsend, xrecv, asend, arecv, ksem, vsem):
        my = lax.axis_index("i")
        left = lax.rem(my + N_DEV - 1, N_DEV)
        right = lax.rem(my + 1, N_DEV)
        h0 = my * HQ_PER

        barrier = pltpu.get_barrier_semaphore()
        for nbr in (left, right):
            pl.semaphore_signal(barrier, inc=1, device_id=(nbr,),
                                device_id_type=pl.DeviceIdType.MESH)
        pl.semaphore_wait(barrier, 2)

        qb = lax.broadcasted_iota(jnp.int32, (SQ, SKV), 0) // BLK
        kb = lax.broadcasted_iota(jnp.int32, (SQ, SKV), 1) // BLK
        mask = (qb == kb) | (kb == 0) | (lax.rem(qb + kb, 3) == 0)
        bias = jnp.where(mask, 0.0, -1e9).astype(jnp.float32)

        def accumulate_partial(xv, b, acc_ref):
            copies = []
            for hh in range(HQ_PER):
                copies.append(pltpu.make_async_copy(
                    k_ref.at[b, :, h0 + hh, :], kbuf.at[hh], ksem))
                copies.append(pltpu.make_async_copy(
                    v_ref.at[b, :, h0 + hh, :], vbuf.at[hh], vsem))
            for c in copies:
                c.start()
            for c in copies:
                c.wait()

            def head_body(hh, _):
                qh = jnp.dot(xv, wq_ref[hh],
                             preferred_element_type=jnp.float32)
                qh = qh.astype(jnp.bfloat16)
                kh = kbuf[hh].astype(jnp.bfloat16)
                s = lax.dot_general(qh, kh, (((1,), (1,)), ((), ())),
                                    preferred_element_type=jnp.float32)
                s = s * SCALE + bias
                m = jnp.max(s, axis=1, keepdims=True)
                w = jnp.exp(s - m)
                den = jnp.sum(w, axis=1, keepdims=True)
                w = (w / den).astype(jnp.bfloat16)
                vh = vbuf[hh].astype(jnp.bfloat16)
                c = jnp.dot(w, vh, preferred_element_type=jnp.float32)
                c = c.astype(jnp.bfloat16)
                acc_ref[:, :] = acc_ref[:, :] + jnp.dot(
                    c, wo_ref[hh], preferred_element_type=jnp.float32)
                return 0

            lax.fori_loop(0, HQ_PER, head_body, 0)

        xbuf[0] = x_ref[:, :]
        accbuf[0] = jnp.zeros((SQ, DM), jnp.float32)
        accumulate_partial(x_ref[:, :], my, accbuf.at[0])

        for h in range(N_DEV - 1):
            rx = pltpu.make_async_remote_copy(
                src_ref=xbuf.at[h], dst_ref=xbuf.at[h + 1],
                send_sem=xsend.at[h], recv_sem=xrecv.at[h + 1],
                device_id=(right,), device_id_type=pl.DeviceIdType.MESH)
            ra = pltpu.make_async_remote_copy(
                src_ref=accbuf.at[h], dst_ref=accbuf.at[h + 1],
                send_sem=asend.at[h], recv_sem=arecv.at[h + 1],
                device_id=(right,), device_id_type=pl.DeviceIdType.MESH)
            rx.start()
            ra.start()
            rx.wait()
            ra.wait()
            b = lax.rem(my - h - 1 + N_DEV, N_DEV)
            accumulate_partial(xbuf[h + 1], b, accbuf.at[h + 1])

        rf = pltpu.make_async_remote_copy(
            src_ref=accbuf.at[N_DEV - 1], dst_ref=accbuf.at[0],
            send_sem=asend.at[N_DEV - 1], recv_sem=arecv.at[0],
            device_id=(right,), device_id_type=pl.DeviceIdType.MESH)
        rf.start()
        rf.wait()
        out_ref[0] = accbuf[0]

    out_shape = jax.ShapeDtypeStruct((1, SQ, DM), jnp.float32)
    return pl.pallas_call(
        body,
        out_shape=out_shape,
        in_specs=[
            pl.BlockSpec(memory_space=pltpu.MemorySpace.VMEM),
            pl.BlockSpec(memory_space=pltpu.MemorySpace.VMEM),
            pl.BlockSpec(memory_space=pltpu.MemorySpace.VMEM),
            pl.BlockSpec(memory_space=pl.ANY),
            pl.BlockSpec(memory_space=pl.ANY),
        ],
        out_specs=pl.BlockSpec(memory_space=pltpu.MemorySpace.VMEM),
        scratch_shapes=[
            pltpu.VMEM((N_DEV, SQ, DM), jnp.bfloat16),
            pltpu.VMEM((N_DEV, SQ, DM), jnp.float32),
            pltpu.VMEM((HQ_PER, SKV, DH), jnp.float32),
            pltpu.VMEM((HQ_PER, SKV, DH), jnp.float32),
            pltpu.SemaphoreType.DMA((N_DEV,)),
            pltpu.SemaphoreType.DMA((N_DEV,)),
            pltpu.SemaphoreType.DMA((N_DEV,)),
            pltpu.SemaphoreType.DMA((N_DEV,)),
            pltpu.SemaphoreType.DMA,
            pltpu.SemaphoreType.DMA,
        ],
        compiler_params=pltpu.CompilerParams(
            collective_id=0,
            vmem_limit_bytes=100 * 1024 * 1024,
        ),
    )(x_my, wq3, wo3, K_ext, V_ext)


# device time: 119355 ns/iter; 1.8243x vs baseline; 1.8243x over previous
import jax
import jax.numpy as jnp
from jax import lax
from jax.experimental import pallas as pl
from jax.experimental.pallas import tpu as pltpu

N_DEV = 4
SQ = 256
SKV = 4096
HQ_PER = 8
DH = 128
DM = 1024
BLK = 64
SCALE = 0.08838834764831843


def kernel(x, Wq, K_ext, V_ext, Wo):
    x_my = x[0].astype(jnp.bfloat16)
    wq3 = Wq.astype(jnp.bfloat16).reshape(1024, HQ_PER, DH)
    wq3 = wq3.transpose(1, 0, 2)
    wo3 = Wo.astype(jnp.bfloat16).reshape(HQ_PER, DH, DM)

    def body(x_ref, wq_ref, wo_ref, k_ref, v_ref, out_ref,
             xbuf, accbuf, ptmp, kbuf, vbuf,
             xsend, xrecv, asend, arecv, ksem, vsem):
        my = lax.axis_index("i")
        left = lax.rem(my + N_DEV - 1, N_DEV)
        right = lax.rem(my + 1, N_DEV)
        h0 = my * HQ_PER

        def start_head_kv(slot, b, head):
            pltpu.make_async_copy(
                k_ref.at[b, :, head, :], kbuf.at[slot], ksem.at[slot]).start()
            pltpu.make_async_copy(
                v_ref.at[b, :, head, :], vbuf.at[slot], vsem.at[slot]).start()

        def wait_head_kv(slot):
            pltpu.make_async_copy(
                k_ref.at[0, :, 0, :], kbuf.at[slot], ksem.at[slot]).wait()
            pltpu.make_async_copy(
                v_ref.at[0, :, 0, :], vbuf.at[slot], vsem.at[slot]).wait()

        def batch_of(h):
            return lax.rem(my - h + N_DEV, N_DEV)

        def make_rx(h):
            return pltpu.make_async_remote_copy(
                src_ref=xbuf.at[h], dst_ref=xbuf.at[h + 1],
                send_sem=xsend.at[h], recv_sem=xrecv.at[h + 1],
                device_id=(right,), device_id_type=pl.DeviceIdType.MESH)

        def make_ra(h):
            return pltpu.make_async_remote_copy(
                src_ref=accbuf.at[h], dst_ref=accbuf.at[(h + 1) % N_DEV],
                send_sem=asend.at[h], recv_sem=arecv.at[(h + 1) % N_DEV],
                device_id=(right,), device_id_type=pl.DeviceIdType.MESH)

        def compute_partial(xv, b, b_next):
            ptmp[:, :] = jnp.zeros((SQ, DM), jnp.float32)

            def head_body(hh, _):
                slot = lax.rem(hh, 2)
                nslot = lax.rem(hh + 1, 2)
                nb = jnp.where(hh == HQ_PER - 1, b_next, b)
                nhead = h0 + lax.rem(hh + 1, HQ_PER)
                start_head_kv(nslot, nb, nhead)
                wait_head_kv(slot)
                qh = jnp.dot(xv, wq_ref[hh],
                             preferred_element_type=jnp.float32)
                qh = qh.astype(jnp.bfloat16)
                kh = kbuf[slot].astype(jnp.bfloat16)
                s = lax.dot_general(qh, kh, (((1,), (1,)), ((), ())),
                                    preferred_element_type=jnp.float32)
                s = s * SCALE + bias
                m = jnp.max(s, axis=1, keepdims=True)
                w = jnp.exp(s - m)
                den = jnp.sum(w, axis=1, keepdims=True)
                w = (w / den).astype(jnp.bfloat16)
                vh = vbuf[slot].astype(jnp.bfloat16)
                c = jnp.dot(w, vh, preferred_element_type=jnp.float32)
                c = c.astype(jnp.bfloat16)
                ptmp[:, :] = ptmp[:, :] + jnp.dot(
                    c, wo_ref[hh], preferred_element_type=jnp.float32)
                return 0

            lax.fori_loop(0, HQ_PER, head_body, 0)

        start_head_kv(0, batch_of(0), h0)

        barrier = pltpu.get_barrier_semaphore()
        for nbr in (left, right):
            pl.semaphore_signal(barrier, inc=1, device_id=(nbr,),
                                device_id_type=pl.DeviceIdType.MESH)
        pl.semaphore_wait(barrier, 2)

        xbuf[0] = x_ref[:, :]
        rx_prev = make_rx(0)
        rx_prev.start()

        qb = lax.broadcasted_iota(jnp.int32, (SQ, SKV), 0) // BLK
        kb = lax.broadcasted_iota(jnp.int32, (SQ, SKV), 1) // BLK
        mask = (qb == kb) | (kb == 0) | (lax.rem(qb + kb, 3) == 0)
        bias = jnp.where(mask, 0.0, -1e9).astype(jnp.float32)

        compute_partial(x_ref[:, :], batch_of(0), batch_of(1))
        accbuf[0] = ptmp[:, :].astype(jnp.bfloat16)
        ra_prev = make_ra(0)
        ra_prev.start()

        for h in range(1, N_DEV):
            rx_prev.wait()
            if h < N_DEV - 1:
                rx_prev = make_rx(h)
                rx_prev.start()
            compute_partial(xbuf[h], batch_of(h), batch_of(h + 1))
            ra_prev.wait()
            accbuf[h] = (accbuf[h][:, :].astype(jnp.float32)
                         + ptmp[:, :]).astype(jnp.bfloat16)
            ra_prev = make_ra(h)
            ra_prev.start()

        wait_head_kv(0)

        ra_prev.wait()
        out_ref[0] = accbuf[0][:, :].astype(jnp.float32)

    out_shape = jax.ShapeDtypeStruct((1, SQ, DM), jnp.float32)
    return pl.pallas_call(
        body,
        out_shape=out_shape,
        in_specs=[
            pl.BlockSpec(memory_space=pltpu.MemorySpace.VMEM),
            pl.BlockSpec(memory_space=pltpu.MemorySpace.VMEM),
            pl.BlockSpec(memory_space=pltpu.MemorySpace.VMEM),
            pl.BlockSpec(memory_space=pl.ANY),
            pl.BlockSpec(memory_space=pl.ANY),
        ],
        out_specs=pl.BlockSpec(memory_space=pltpu.MemorySpace.VMEM),
        scratch_shapes=[
            pltpu.VMEM((N_DEV, SQ, DM), jnp.bfloat16),
            pltpu.VMEM((N_DEV, SQ, DM), jnp.bfloat16),
            pltpu.VMEM((SQ, DM), jnp.float32),
            pltpu.VMEM((2, SKV, DH), jnp.float32),
            pltpu.VMEM((2, SKV, DH), jnp.float32),
            pltpu.SemaphoreType.DMA((N_DEV,)),
            pltpu.SemaphoreType.DMA((N_DEV,)),
            pltpu.SemaphoreType.DMA((N_DEV,)),
            pltpu.SemaphoreType.DMA((N_DEV,)),
            pltpu.SemaphoreType.DMA((2,)),
            pltpu.SemaphoreType.DMA((2,)),
        ],
        compiler_params=pltpu.CompilerParams(
            collective_id=0,
            vmem_limit_bytes=60 * 1024 * 1024,
        ),
    )(x_my, wq3, wo3, K_ext, V_ext)
